# baseline (speedup 1.0000x reference)
import functools

import jax
import jax.numpy as jnp
from jax import lax
from jax.experimental import pallas as pl
from jax.experimental.pallas import tpu as pltpu
from jax.experimental.pallas import tpu_sc as plsc

B, S, F = 1024, 50, 26
NSPARSE = F - 1
EMB = 32
OUT_W = 1 + NSPARSE * EMB
S_PAD = 56
W_PAD = 896
H0 = 24
H1 = S - H0
R1 = S_PAD - H0
IPB = S * NSPARSE

NC, NS = 2, 16
NW = NC * NS
TRIPS = B // NW
L = 16

IDS_ROWS = B * S * NSPARSE // 128
IDS_WIN = 11
DV_ROWS = B * S // 128
DV_WIN = 2


def _sc_body(ids_hbm, dense_hbm, table_hbm, out_hbm,
             idsW, dvW, idxT, g0, g1, a0, a1,
             sg0, sg1, so0, so1, si):
    cid = lax.axis_index("c")
    sid = lax.axis_index("s")
    wid = sid * NC + cid
    iota = lax.iota(jnp.int32, L)
    zeros = jnp.zeros((L,), jnp.int32)

    def fire_loads(i):
        b = i * NW + wid
        r0 = jnp.minimum((b * IPB) >> 7, IDS_ROWS - IDS_WIN)
        r0d = jnp.minimum((b * S) >> 7, DV_ROWS - DV_WIN)
        par = lax.rem(i, 2)
        pltpu.async_copy(ids_hbm.at[pl.ds(r0, IDS_WIN), :], idsW.at[par], si)
        pltpu.async_copy(dense_hbm.at[pl.ds(r0d, DV_WIN), :], dvW.at[par], si)

    def wait_loads():
        pltpu.make_async_copy(ids_hbm.at[pl.ds(0, IDS_WIN), :],
                              idsW.at[0], si).wait()
        pltpu.make_async_copy(dense_hbm.at[pl.ds(0, DV_WIN), :],
                              dvW.at[0], si).wait()
    blocks = ((0, 0, H0), (1, H0, H1))
    posv = {(blk, off): (iota + off + base) * NSPARSE
            for blk, base, n in blocks for off in (0, n - L)}

    def assemble(gb, am, base, n, od, dvC):
        def pos(q, c2):
            for u in (0, 1):
                p = 2 * q + u
                for r in range(NSPARSE):
                    row = r * n + p
                    am[p, pl.ds(1 + r * EMB, L)] = gb[row, pl.ds(0, L)]
                    am[p, pl.ds(1 + r * EMB + L, L)] = gb[row, pl.ds(L, L)]
            return c2

        lax.fori_loop(0, n // 2, pos, 0)
        for off in (0, n - L):
            fl = iota + off + base + od
            vals = plsc.load_gather(dvC, [fl >> 7, fl & 127])
            plsc.store_scatter(am, [iota + off, zeros], vals)

    def chunk(i, carry):
        b = i * NW + wid
        r0 = jnp.minimum((b * IPB) >> 7, IDS_ROWS - IDS_WIN)
        o = b * IPB - (r0 << 7)
        r0d = jnp.minimum((b * S) >> 7, DV_ROWS - DV_WIN)
        od = b * S - (r0d << 7)
        par = lax.rem(i, 2)
        wait_loads()
        idsC = idsW.at[par]
        dvC = dvW.at[par]

        @pl.when(i + 1 < TRIPS)
        def _():
            fire_loads(i + 1)

        for j in range(NSPARSE):
            for blk, base, n in blocks:
                for off in (0, n - L):
                    fl = posv[(blk, off)] + (o + j)
                    v = plsc.load_gather(idsC, [fl >> 7, fl & 127])
                    idxT[2 * j + blk, pl.ds(off, L)] = v
        gath0 = [
            pltpu.async_copy(table_hbm.at[idxT.at[2 * j, pl.ds(0, H0)]],
                             g0.at[pl.ds(j * H0, H0), :], sg0)
            for j in range(NSPARSE)
        ]
        gath1 = [
            pltpu.async_copy(table_hbm.at[idxT.at[2 * j + 1]],
                             g1.at[pl.ds(j * H1, H1), :], sg1)
            for j in range(NSPARSE)
        ]

        @pl.when(i > 0)
        def _():
            pltpu.make_async_copy(a0, out_hbm.at[b, pl.ds(0, H0), :],
                                  so0).wait()

        for cp in gath0:
            cp.wait()
        assemble(g0, a0, 0, H0, od, dvC)
        pltpu.async_copy(a0, out_hbm.at[b, pl.ds(0, H0), :], so0)

        @pl.when(i > 0)
        def _():
            pltpu.make_async_copy(a1, out_hbm.at[b, pl.ds(H0, R1), :],
                                  so1).wait()

        for cp in gath1:
            cp.wait()
        assemble(g1, a1, H0, H1, od, dvC)
        pltpu.async_copy(a1, out_hbm.at[b, pl.ds(H0, R1), :], so1)
        return carry

    fire_loads(0)
    lax.fori_loop(0, TRIPS, chunk, 0)
    pltpu.make_async_copy(a0, out_hbm.at[0, pl.ds(0, H0), :], so0).wait()
    pltpu.make_async_copy(a1, out_hbm.at[0, pl.ds(H0, R1), :], so1).wait()


@jax.jit
def _sc_call(ids_g, dense, table):
    return pl.kernel(
        _sc_body,
        out_type=jax.ShapeDtypeStruct((B, S_PAD, W_PAD), jnp.float32),
        mesh=plsc.VectorSubcoreMesh(
            core_axis_name="c", subcore_axis_name="s",
            num_cores=NC, num_subcores=NS,
        ),
        scratch_types=[
            pltpu.VMEM((2, IDS_WIN, 128), jnp.int32),
            pltpu.VMEM((2, DV_WIN, 128), jnp.float32),
            pltpu.VMEM((2 * NSPARSE, H1), jnp.int32),
            pltpu.VMEM((H0 * NSPARSE, EMB), jnp.float32),
            pltpu.VMEM((H1 * NSPARSE, EMB), jnp.float32),
            pltpu.VMEM((H0, W_PAD), jnp.float32),
            pltpu.VMEM((R1, W_PAD), jnp.float32),
            pltpu.SemaphoreType.DMA,
            pltpu.SemaphoreType.DMA,
            pltpu.SemaphoreType.DMA,
            pltpu.SemaphoreType.DMA,
            pltpu.SemaphoreType.DMA,
        ],
        compiler_params=pltpu.CompilerParams(
            use_tc_tiling_on_sc=False, needs_layout_passes=False),
    )(ids_g, dense, table)


def kernel(x, emb0):
    ids_g = x[:, :, 1:].astype(jnp.int32).reshape(IDS_ROWS, 128)
    dense = x[:, :, 0].reshape(DV_ROWS, 128)
    out = _sc_call(ids_g, dense, emb0)
    return out[:, :S, :OUT_W]


# speedup vs baseline: 1.0046x; 1.0046x over previous
import functools

import jax
import jax.numpy as jnp
from jax import lax
from jax.experimental import pallas as pl
from jax.experimental.pallas import tpu as pltpu
from jax.experimental.pallas import tpu_sc as plsc

B, S, F = 1024, 50, 26
NSPARSE = F - 1
EMB = 32
OUT_W = 1 + NSPARSE * EMB
S_PAD = 56
W_PAD = 896
H0 = 24
H1 = S - H0
R1 = S_PAD - H0
IPB = S * NSPARSE

NC, NS = 2, 16
NW = NC * NS
TRIPS = B // NW
L = 16

IDS_ROWS = B * S * NSPARSE // 128
IDS_WIN = 11
DV_ROWS = B * S // 128
DV_WIN = 2


def _sc_body(ids_hbm, dense_hbm, table_hbm, out_hbm,
             idsW, dvW, idxT, g0, g1, a0, a1,
             sg0, sg1, so0, so1, si):
    cid = lax.axis_index("c")
    sid = lax.axis_index("s")
    wid = sid * NC + cid
    iota = lax.iota(jnp.int32, L)
    zeros = jnp.zeros((L,), jnp.int32)

    def fire_loads(i):
        b = i * NW + wid
        r0 = jnp.minimum((b * IPB) >> 7, IDS_ROWS - IDS_WIN)
        r0d = jnp.minimum((b * S) >> 7, DV_ROWS - DV_WIN)
        par = lax.rem(i, 2)
        pltpu.async_copy(ids_hbm.at[pl.ds(r0, IDS_WIN), :], idsW.at[par], si)
        pltpu.async_copy(dense_hbm.at[pl.ds(r0d, DV_WIN), :], dvW.at[par], si)

    def wait_loads():
        pltpu.make_async_copy(ids_hbm.at[pl.ds(0, IDS_WIN), :],
                              idsW.at[0], si).wait()
        pltpu.make_async_copy(dense_hbm.at[pl.ds(0, DV_WIN), :],
                              dvW.at[0], si).wait()
    blocks = ((0, 0, H0), (1, H0, H1))
    posv = {(blk, off): (iota + off + base) * NSPARSE
            for blk, base, n in blocks for off in (0, n - L)}

    def assemble(gb, am, base, n, od, dvC):
        def pos(p, c2):
            for r in range(NSPARSE):
                row = r * n + p
                am[p, pl.ds(1 + r * EMB, L)] = gb[row, pl.ds(0, L)]
                am[p, pl.ds(1 + r * EMB + L, L)] = gb[row, pl.ds(L, L)]
            return c2

        lax.fori_loop(0, n, pos, 0)
        for off in (0, n - L):
            fl = iota + off + base + od
            vals = plsc.load_gather(dvC, [fl >> 7, fl & 127])
            plsc.store_scatter(am, [iota + off, zeros], vals)

    def chunk(i, carry):
        b = i * NW + wid
        r0 = jnp.minimum((b * IPB) >> 7, IDS_ROWS - IDS_WIN)
        o = b * IPB - (r0 << 7)
        r0d = jnp.minimum((b * S) >> 7, DV_ROWS - DV_WIN)
        od = b * S - (r0d << 7)
        par = lax.rem(i, 2)
        wait_loads()
        idsC = idsW.at[par]
        dvC = dvW.at[par]

        @pl.when(i + 1 < TRIPS)
        def _():
            fire_loads(i + 1)

        for j in range(NSPARSE):
            for blk, base, n in blocks:
                for off in (0, n - L):
                    fl = posv[(blk, off)] + (o + j)
                    v = plsc.load_gather(idsC, [fl >> 7, fl & 127])
                    idxT[2 * j + blk, pl.ds(off, L)] = v
        gath0 = [
            pltpu.async_copy(table_hbm.at[idxT.at[2 * j, pl.ds(0, H0)]],
                             g0.at[pl.ds(j * H0, H0), :], sg0)
            for j in range(NSPARSE)
        ]
        gath1 = [
            pltpu.async_copy(table_hbm.at[idxT.at[2 * j + 1]],
                             g1.at[pl.ds(j * H1, H1), :], sg1)
            for j in range(NSPARSE)
        ]

        @pl.when(i > 0)
        def _():
            pltpu.make_async_copy(a0, out_hbm.at[b, pl.ds(0, H0), :],
                                  so0).wait()

        for cp in gath0:
            cp.wait()
        assemble(g0, a0, 0, H0, od, dvC)
        pltpu.async_copy(a0, out_hbm.at[b, pl.ds(0, H0), :], so0)

        @pl.when(i > 0)
        def _():
            pltpu.make_async_copy(a1, out_hbm.at[b, pl.ds(H0, R1), :],
                                  so1).wait()

        for cp in gath1:
            cp.wait()
        assemble(g1, a1, H0, H1, od, dvC)
        pltpu.async_copy(a1, out_hbm.at[b, pl.ds(H0, R1), :], so1)
        return carry

    fire_loads(0)
    lax.fori_loop(0, TRIPS, chunk, 0)
    pltpu.make_async_copy(a0, out_hbm.at[0, pl.ds(0, H0), :], so0).wait()
    pltpu.make_async_copy(a1, out_hbm.at[0, pl.ds(H0, R1), :], so1).wait()


@jax.jit
def _sc_call(ids_g, dense, table):
    return pl.kernel(
        _sc_body,
        out_type=jax.ShapeDtypeStruct((B, S_PAD, W_PAD), jnp.float32),
        mesh=plsc.VectorSubcoreMesh(
            core_axis_name="c", subcore_axis_name="s",
            num_cores=NC, num_subcores=NS,
        ),
        scratch_types=[
            pltpu.VMEM((2, IDS_WIN, 128), jnp.int32),
            pltpu.VMEM((2, DV_WIN, 128), jnp.float32),
            pltpu.VMEM((2 * NSPARSE, H1), jnp.int32),
            pltpu.VMEM((H0 * NSPARSE, EMB), jnp.float32),
            pltpu.VMEM((H1 * NSPARSE, EMB), jnp.float32),
            pltpu.VMEM((H0, W_PAD), jnp.float32),
            pltpu.VMEM((R1, W_PAD), jnp.float32),
            pltpu.SemaphoreType.DMA,
            pltpu.SemaphoreType.DMA,
            pltpu.SemaphoreType.DMA,
            pltpu.SemaphoreType.DMA,
            pltpu.SemaphoreType.DMA,
        ],
        compiler_params=pltpu.CompilerParams(
            use_tc_tiling_on_sc=False, needs_layout_passes=False),
    )(ids_g, dense, table)


def kernel(x, emb0):
    ids_g = x[:, :, 1:].astype(jnp.int32).reshape(IDS_ROWS, 128)
    dense = x[:, :, 0].reshape(DV_ROWS, 128)
    out = _sc_call(ids_g, dense, emb0)
    return out[:, :S, :OUT_W]
